# baseline (speedup 1.0000x reference)
import functools

import jax
import jax.numpy as jnp
from jax import lax
from jax.experimental import pallas as pl
from jax.experimental.pallas import tpu as pltpu
from jax.experimental.pallas import tpu_sc as plsc

B, L = 16384, 200
V, D = 1_000_000, 16
LP = 256
NC, NS = 2, 16
NW = NC * NS
ROWS_PER_W = B // NW
CHUNK_ROWS = 64
N_CHUNKS = ROWS_PER_W // CHUNK_ROWS
IDX_PER_CHUNK = CHUNK_ROWS * L


_CB = 262144


def _tc_head_body(tt_ref, w_ref, b_ref, out_ref):
    s = jnp.dot(w_ref[...], tt_ref[...], preferred_element_type=jnp.float32)
    out_ref[...] = s[0] + b_ref[0, 0]


def _precompute_scores(table, W, b):
    tt = table.T
    wc = (W * (1.0 / L)).astype(jnp.float32)
    b2 = (b * (1.0 / L)).reshape(1, 1)
    return pl.pallas_call(
        _tc_head_body,
        grid=(pl.cdiv(V, _CB),),
        in_specs=[
            pl.BlockSpec((16, _CB), lambda i: (0, i)),
            pl.BlockSpec((1, 16), lambda i: (0, 0)),
            pl.BlockSpec((1, 1), lambda i: (0, 0), memory_space=pltpu.SMEM),
        ],
        out_specs=pl.BlockSpec((_CB,), lambda i: (i,)),
        out_shape=jax.ShapeDtypeStruct((V,), jnp.float32),
    )(tt, wc, b2)


def _sc_body(t_hbm, xf_hbm, g_hbm, t_sp, idx_v, vals_v, sem_g, sem_w):
    sid = lax.axis_index("s")
    wid = sid * NC + lax.axis_index("c")
    w_base = wid * (N_CHUNKS * IDX_PER_CHUNK)

    _SB = 10000
    _NB = V // _SB
    n_mine = -(-_NB // NS)

    def _wait_out(k):
        pltpu.make_async_copy(
            t_hbm.at[pl.ds(0, _SB)], vals_v[k % 2].at[pl.ds(0, _SB)],
            sem_w[k % 2],
        ).wait()

    for k in range(n_mine):
        blk = k * NS + sid

        @pl.when(blk < _NB)
        def _(blk=blk, k=k):
            if k >= 2:
                _wait_out(k - 2)
            pltpu.async_copy(
                t_hbm.at[pl.ds(blk * _SB, _SB)],
                vals_v[k % 2].at[pl.ds(0, _SB)],
                sem_g[k % 2],
            ).wait()
            pltpu.async_copy(
                vals_v[k % 2].at[pl.ds(0, _SB)],
                t_sp.at[pl.ds(blk * _SB, _SB)],
                sem_w[k % 2],
            )

    for k in range(max(0, n_mine - 2), n_mine):
        blk = k * NS + sid

        @pl.when(blk < _NB)
        def _(blk=blk, k=k):
            _wait_out(k)

    plsc.subcore_barrier()

    def stage(c):
        pltpu.sync_copy(
            xf_hbm.at[pl.ds(w_base + c * IDX_PER_CHUNK, IDX_PER_CHUNK)],
            idx_v[c % 2],
        )
        return pltpu.async_copy(t_sp.at[idx_v[c % 2]], vals_v[c % 2], sem_g[c % 2])

    def write(c):
        return pltpu.async_copy(
            vals_v[c % 2],
            g_hbm.at[pl.ds(w_base + c * IDX_PER_CHUNK, IDX_PER_CHUNK)],
            sem_w[c % 2],
        )

    wr = [None, None]
    cp = stage(0)
    for c in range(N_CHUNKS):
        if c + 1 < N_CHUNKS:
            if wr[(c + 1) % 2] is not None:
                wr[(c + 1) % 2].wait()
                wr[(c + 1) % 2] = None
            nxt = stage(c + 1)
        else:
            nxt = None
        cp.wait()
        cp = nxt
        wr[c % 2] = write(c)
    for w in wr:
        if w is not None:
            w.wait()


@functools.partial(
    pl.kernel,
    out_type=jax.ShapeDtypeStruct((B * L,), jnp.float32),
    mesh=plsc.VectorSubcoreMesh(
        core_axis_name="c", subcore_axis_name="s", num_cores=NC, num_subcores=NS
    ),
    scratch_types=[
        pltpu.VMEM_SHARED((V,), jnp.float32),
        [pltpu.VMEM((IDX_PER_CHUNK,), jnp.int32) for _ in range(2)],
        [pltpu.VMEM((IDX_PER_CHUNK,), jnp.float32) for _ in range(2)],
        [pltpu.SemaphoreType.DMA for _ in range(2)],
        [pltpu.SemaphoreType.DMA for _ in range(2)],
    ],
)
def _sc_gather(t_hbm, xf_hbm, g_hbm, t_sp, idx_v, vals_v, sem_g, sem_w):
    _sc_body(t_hbm, xf_hbm, g_hbm, t_sp, idx_v, vals_v, sem_g, sem_w)


_SLAB = 25
_NG = L // _SLAB


def _tc_pool_body(g_ref, out_ref):
    i = pl.program_id(0)
    part = jnp.sum(g_ref[...].reshape(_SLAB, B), axis=0)

    @pl.when(i == 0)
    def _():
        out_ref[...] = part

    @pl.when(i > 0)
    def _():
        out_ref[...] = out_ref[...] + part

    @pl.when(i == _NG - 1)
    def _():
        s = out_ref[...]
        y = 1.0 / (1.0 + jnp.exp(-s))
        out_ref[...] = jnp.round(y * 10000.0) / 10000.0


def _pool_scores(g):
    return pl.pallas_call(
        _tc_pool_body,
        grid=(_NG,),
        in_specs=[pl.BlockSpec((_SLAB * B,), lambda i: (i,))],
        out_specs=pl.BlockSpec((B,), lambda i: (0,)),
        out_shape=jax.ShapeDtypeStruct((B,), jnp.float32),
    )(g)


def kernel(x, table, W, b):
    t = _precompute_scores(table, W, b)
    xf = x.T.astype(jnp.int32).reshape(B * L)
    g = _sc_gather(t, xf)
    out = _pool_scores(g)
    return out.reshape(B, 1)


# speedup vs baseline: 1.0065x; 1.0065x over previous
import functools

import jax
import jax.numpy as jnp
from jax import lax
from jax.experimental import pallas as pl
from jax.experimental.pallas import tpu as pltpu
from jax.experimental.pallas import tpu_sc as plsc

B, L = 16384, 200
V, D = 1_000_000, 16
NC, NS = 2, 16
NW = NC * NS
IDX_PER_CHUNK = 12800
N_CHUNKS = B * L // NW // IDX_PER_CHUNK


_CB = 131072


def _tc_head_body(tt_ref, w_ref, b_ref, out_ref):
    s = jnp.dot(w_ref[...], tt_ref[...], preferred_element_type=jnp.float32)
    out_ref[...] = s[0] + b_ref[0, 0]


def _precompute_scores(table, W, b):
    tt = table.T
    wc = (W * (1.0 / L)).astype(jnp.float32)
    b2 = (b * (1.0 / L)).reshape(1, 1)
    return pl.pallas_call(
        _tc_head_body,
        grid=(pl.cdiv(V, _CB),),
        in_specs=[
            pl.BlockSpec((16, _CB), lambda i: (0, i)),
            pl.BlockSpec((1, 16), lambda i: (0, 0)),
            pl.BlockSpec((1, 1), lambda i: (0, 0), memory_space=pltpu.SMEM),
        ],
        out_specs=pl.BlockSpec((_CB,), lambda i: (i,)),
        out_shape=jax.ShapeDtypeStruct((V,), jnp.float32),
    )(tt, wc, b2)


def _sc_body(t_hbm, xf_hbm, g_hbm, t_sp, idx_v, vals_v, sem_g, sem_w):
    sid = lax.axis_index("s")
    wid = sid * NC + lax.axis_index("c")
    w_base = wid * (N_CHUNKS * IDX_PER_CHUNK)

    _SB = 10000
    _NB = V // _SB
    n_mine = -(-_NB // NS)

    def _wait_out(k):
        pltpu.make_async_copy(
            t_hbm.at[pl.ds(0, _SB)], vals_v[k % 2].at[pl.ds(0, _SB)],
            sem_w[k % 2],
        ).wait()

    for k in range(n_mine):
        blk = k * NS + sid

        @pl.when(blk < _NB)
        def _(blk=blk, k=k):
            if k >= 2:
                _wait_out(k - 2)
            pltpu.async_copy(
                t_hbm.at[pl.ds(blk * _SB, _SB)],
                vals_v[k % 2].at[pl.ds(0, _SB)],
                sem_g[k % 2],
            ).wait()
            pltpu.async_copy(
                vals_v[k % 2].at[pl.ds(0, _SB)],
                t_sp.at[pl.ds(blk * _SB, _SB)],
                sem_w[k % 2],
            )

    for k in range(max(0, n_mine - 2), n_mine):
        blk = k * NS + sid

        @pl.when(blk < _NB)
        def _(blk=blk, k=k):
            _wait_out(k)

    plsc.subcore_barrier()

    def stage(c):
        pltpu.sync_copy(
            xf_hbm.at[pl.ds(w_base + c * IDX_PER_CHUNK, IDX_PER_CHUNK)],
            idx_v[c % 2],
        )
        return pltpu.async_copy(t_sp.at[idx_v[c % 2]], vals_v[c % 2], sem_g[c % 2])

    def write(c):
        return pltpu.async_copy(
            vals_v[c % 2],
            g_hbm.at[pl.ds(w_base + c * IDX_PER_CHUNK, IDX_PER_CHUNK)],
            sem_w[c % 2],
        )

    wr = [None, None]
    cp = stage(0)
    for c in range(N_CHUNKS):
        if c + 1 < N_CHUNKS:
            if wr[(c + 1) % 2] is not None:
                wr[(c + 1) % 2].wait()
                wr[(c + 1) % 2] = None
            nxt = stage(c + 1)
        else:
            nxt = None
        cp.wait()
        cp = nxt
        wr[c % 2] = write(c)
    for w in wr:
        if w is not None:
            w.wait()


@functools.partial(
    pl.kernel,
    out_type=jax.ShapeDtypeStruct((B * L,), jnp.float32),
    mesh=plsc.VectorSubcoreMesh(
        core_axis_name="c", subcore_axis_name="s", num_cores=NC, num_subcores=NS
    ),
    scratch_types=[
        pltpu.VMEM_SHARED((V,), jnp.float32),
        [pltpu.VMEM((IDX_PER_CHUNK,), jnp.int32) for _ in range(2)],
        [pltpu.VMEM((IDX_PER_CHUNK,), jnp.float32) for _ in range(2)],
        [pltpu.SemaphoreType.DMA for _ in range(2)],
        [pltpu.SemaphoreType.DMA for _ in range(2)],
    ],
)
def _sc_gather(t_hbm, xf_hbm, g_hbm, t_sp, idx_v, vals_v, sem_g, sem_w):
    _sc_body(t_hbm, xf_hbm, g_hbm, t_sp, idx_v, vals_v, sem_g, sem_w)


_SLAB = 25
_NG = L // _SLAB


def _tc_pool_body(g_ref, out_ref):
    i = pl.program_id(0)
    part = jnp.sum(g_ref[...].reshape(_SLAB, B), axis=0)

    @pl.when(i == 0)
    def _():
        out_ref[...] = part

    @pl.when(i > 0)
    def _():
        out_ref[...] = out_ref[...] + part

    @pl.when(i == _NG - 1)
    def _():
        s = out_ref[...]
        y = 1.0 / (1.0 + jnp.exp(-s))
        out_ref[...] = jnp.round(y * 10000.0) / 10000.0


def _pool_scores(g):
    return pl.pallas_call(
        _tc_pool_body,
        grid=(_NG,),
        in_specs=[pl.BlockSpec((_SLAB * B,), lambda i: (i,))],
        out_specs=pl.BlockSpec((B,), lambda i: (0,)),
        out_shape=jax.ShapeDtypeStruct((B,), jnp.float32),
    )(g)


def kernel(x, table, W, b):
    t = _precompute_scores(table, W, b)
    xf = x.T.astype(jnp.int32).reshape(B * L)
    g = _sc_gather(t, xf)
    out = _pool_scores(g)
    return out.reshape(B, 1)
